# baseline (device time: 30444 ns/iter reference)
import jax
import jax.numpy as jnp
from jax import lax
from jax.experimental import pallas as pl
from jax.experimental.pallas import tpu as pltpu

N_DEV = 4


def kernel(A, B):
    m_per, k = A.shape
    _, n = B.shape

    def body(a_ref, b_ref, out_ref, comm_ref, send_sems, recv_sems):
        my_pos = lax.axis_index("i")
        left = (my_pos - 1) % N_DEV
        right = (my_pos + 1) % N_DEV

        barrier_sem = pltpu.get_barrier_semaphore()
        for nbr in [left, right]:
            pl.semaphore_signal(
                barrier_sem, inc=1,
                device_id=(nbr,), device_id_type=pl.DeviceIdType.MESH,
            )
        pl.semaphore_wait(barrier_sem, 2)

        comm_ref[0] = a_ref[...]
        out_ref[pl.ds(my_pos * m_per, m_per), :] = jnp.dot(
            a_ref[...], b_ref[...], preferred_element_type=jnp.float32
        )

        for h in range(N_DEV - 1):
            send_slot = h % 2
            recv_slot = (h + 1) % 2
            rdma = pltpu.make_async_remote_copy(
                src_ref=comm_ref.at[send_slot],
                dst_ref=comm_ref.at[recv_slot],
                send_sem=send_sems.at[send_slot],
                recv_sem=recv_sems.at[recv_slot],
                device_id=(right,),
                device_id_type=pl.DeviceIdType.MESH,
            )
            rdma.start()
            rdma.wait()

            origin = (my_pos - h - 1) % N_DEV
            out_ref[pl.ds(origin * m_per, m_per), :] = jnp.dot(
                comm_ref[recv_slot], b_ref[...],
                preferred_element_type=jnp.float32,
            )

    return pl.pallas_call(
        body,
        out_shape=jax.ShapeDtypeStruct((N_DEV * m_per, n), jnp.float32),
        in_specs=[
            pl.BlockSpec(memory_space=pltpu.VMEM),
            pl.BlockSpec(memory_space=pltpu.VMEM),
        ],
        out_specs=pl.BlockSpec(memory_space=pltpu.VMEM),
        scratch_shapes=[
            pltpu.VMEM((2, m_per, k), jnp.float32),
            pltpu.SemaphoreType.DMA((2,)),
            pltpu.SemaphoreType.DMA((2,)),
        ],
        compiler_params=pltpu.CompilerParams(collective_id=0),
    )(A, B)


# device time: 19331 ns/iter; 1.5749x vs baseline; 1.5749x over previous
import jax
import jax.numpy as jnp
from jax import lax
from jax.experimental import pallas as pl
from jax.experimental.pallas import tpu as pltpu

N_DEV = 4


def kernel(A, B):
    m_per, k = A.shape
    _, n = B.shape
    half = m_per // 2

    def body(a_ref, b_ref, out_ref, from_l, from_r, from_opp,
             send_sems, recv_sems):
        my_pos = lax.axis_index("i")
        left = (my_pos - 1) % N_DEV
        right = (my_pos + 1) % N_DEV
        opp = (my_pos + 2) % N_DEV

        barrier_sem = pltpu.get_barrier_semaphore()
        for nbr in [left, right]:
            pl.semaphore_signal(
                barrier_sem, inc=1,
                device_id=(nbr,), device_id_type=pl.DeviceIdType.MESH,
            )
        pl.semaphore_wait(barrier_sem, 2)

        send_right = pltpu.make_async_remote_copy(
            src_ref=a_ref, dst_ref=from_l,
            send_sem=send_sems.at[0], recv_sem=recv_sems.at[0],
            device_id=(right,), device_id_type=pl.DeviceIdType.MESH,
        )
        send_left = pltpu.make_async_remote_copy(
            src_ref=a_ref, dst_ref=from_r,
            send_sem=send_sems.at[1], recv_sem=recv_sems.at[1],
            device_id=(left,), device_id_type=pl.DeviceIdType.MESH,
        )
        send_right.start()
        send_left.start()

        out_ref[pl.ds(my_pos * m_per, m_per), :] = jnp.dot(
            a_ref[...], b_ref[...], preferred_element_type=jnp.float32
        )

        send_right.wait_recv()
        relay_right = pltpu.make_async_remote_copy(
            src_ref=from_l.at[pl.ds(0, half)],
            dst_ref=from_opp.at[pl.ds(0, half)],
            send_sem=send_sems.at[2], recv_sem=recv_sems.at[2],
            device_id=(right,), device_id_type=pl.DeviceIdType.MESH,
        )
        relay_right.start()
        out_ref[pl.ds(left * m_per, m_per), :] = jnp.dot(
            from_l[...], b_ref[...], preferred_element_type=jnp.float32
        )

        send_left.wait_recv()
        relay_left = pltpu.make_async_remote_copy(
            src_ref=from_r.at[pl.ds(half, half)],
            dst_ref=from_opp.at[pl.ds(half, half)],
            send_sem=send_sems.at[3], recv_sem=recv_sems.at[3],
            device_id=(left,), device_id_type=pl.DeviceIdType.MESH,
        )
        relay_left.start()
        out_ref[pl.ds(right * m_per, m_per), :] = jnp.dot(
            from_r[...], b_ref[...], preferred_element_type=jnp.float32
        )

        relay_right.wait_recv()
        relay_left.wait_recv()
        out_ref[pl.ds(opp * m_per, m_per), :] = jnp.dot(
            from_opp[...], b_ref[...], preferred_element_type=jnp.float32
        )

        send_right.wait_send()
        send_left.wait_send()
        relay_right.wait_send()
        relay_left.wait_send()

    return pl.pallas_call(
        body,
        out_shape=jax.ShapeDtypeStruct((N_DEV * m_per, n), jnp.float32),
        in_specs=[
            pl.BlockSpec(memory_space=pltpu.VMEM),
            pl.BlockSpec(memory_space=pltpu.VMEM),
        ],
        out_specs=pl.BlockSpec(memory_space=pltpu.VMEM),
        scratch_shapes=[
            pltpu.VMEM((m_per, k), jnp.float32),
            pltpu.VMEM((m_per, k), jnp.float32),
            pltpu.VMEM((m_per, k), jnp.float32),
            pltpu.SemaphoreType.DMA((4,)),
            pltpu.SemaphoreType.DMA((4,)),
        ],
        compiler_params=pltpu.CompilerParams(collective_id=0),
    )(A, B)


# device time: 17807 ns/iter; 1.7097x vs baseline; 1.0856x over previous
import jax
import jax.numpy as jnp
from jax import lax
from jax.experimental import pallas as pl
from jax.experimental.pallas import tpu as pltpu

N_DEV = 4


def kernel(A, B):
    m_per, k = A.shape
    _, n = B.shape
    half = m_per // 2

    def body(a_ref, b_ref, out_ref, from_l, from_r, from_opp,
             send_sems, recv_sems):
        my_pos = lax.axis_index("i")
        left = (my_pos - 1) % N_DEV
        right = (my_pos + 1) % N_DEV
        opp = (my_pos + 2) % N_DEV

        top = pl.ds(0, half)
        bot = pl.ds(half, half)

        def copy(src, dst, s_sem, r_sem, target):
            return pltpu.make_async_remote_copy(
                src_ref=src, dst_ref=dst,
                send_sem=send_sems.at[s_sem], recv_sem=recv_sems.at[r_sem],
                device_id=(target,), device_id_type=pl.DeviceIdType.MESH,
            )

        barrier_sem = pltpu.get_barrier_semaphore()
        for nbr in [left, right]:
            pl.semaphore_signal(
                barrier_sem, inc=1,
                device_id=(nbr,), device_id_type=pl.DeviceIdType.MESH,
            )
        pl.semaphore_wait(barrier_sem, 2)

        sr_top = copy(a_ref.at[top], from_l.at[top], 0, 0, right)
        sr_bot = copy(a_ref.at[bot], from_l.at[bot], 1, 1, right)
        sl_bot = copy(a_ref.at[bot], from_r.at[bot], 2, 2, left)
        sl_top = copy(a_ref.at[top], from_r.at[top], 3, 3, left)
        sr_top.start()
        sl_bot.start()
        sr_bot.start()
        sl_top.start()

        out_ref[pl.ds(my_pos * m_per, m_per), :] = jnp.dot(
            a_ref[...], b_ref[...], preferred_element_type=jnp.float32
        )

        relay_r = copy(from_l.at[top], from_opp.at[top], 4, 4, right)
        relay_l = copy(from_r.at[bot], from_opp.at[bot], 5, 5, left)
        sr_top.wait_recv()
        relay_r.start()
        sl_bot.wait_recv()
        relay_l.start()

        sr_bot.wait_recv()
        out_ref[pl.ds(left * m_per, m_per), :] = jnp.dot(
            from_l[...], b_ref[...], preferred_element_type=jnp.float32
        )
        sl_top.wait_recv()
        out_ref[pl.ds(right * m_per, m_per), :] = jnp.dot(
            from_r[...], b_ref[...], preferred_element_type=jnp.float32
        )

        relay_r.wait_recv()
        relay_l.wait_recv()
        out_ref[pl.ds(opp * m_per, m_per), :] = jnp.dot(
            from_opp[...], b_ref[...], preferred_element_type=jnp.float32
        )

        for r in [sr_top, sr_bot, sl_bot, sl_top, relay_r, relay_l]:
            r.wait_send()

    return pl.pallas_call(
        body,
        out_shape=jax.ShapeDtypeStruct((N_DEV * m_per, n), jnp.float32),
        in_specs=[
            pl.BlockSpec(memory_space=pltpu.VMEM),
            pl.BlockSpec(memory_space=pltpu.VMEM),
        ],
        out_specs=pl.BlockSpec(memory_space=pltpu.VMEM),
        scratch_shapes=[
            pltpu.VMEM((m_per, k), jnp.float32),
            pltpu.VMEM((m_per, k), jnp.float32),
            pltpu.VMEM((m_per, k), jnp.float32),
            pltpu.SemaphoreType.DMA((6,)),
            pltpu.SemaphoreType.DMA((6,)),
        ],
        compiler_params=pltpu.CompilerParams(collective_id=0),
    )(A, B)
